# baseline (speedup 1.0000x reference)
import jax
import jax.numpy as jnp
from jax.experimental import pallas as pl
from jax.experimental.pallas import tpu as pltpu

_BEAM = 10
_FC = 256

_INTERPRET = False


def _mmt(a, b):
    return jax.lax.dot_general(a, b, (((1,), (1,)), ((), ())),
                               preferred_element_type=jnp.float32)


def _mm(a, b):
    return jax.lax.dot_general(a, b, (((1,), (0,)), ((), ())),
                               preferred_element_type=jnp.float32)


def _prove_kernel(nb_ref, rel_ref, a1_ref, a2_ref, fr_ref, fa1_ref, fa2_ref,
                  ent_ref, w1_ref, w2_ref, out_ref):
    b = pl.program_id(0)
    F = fr_ref.shape[1]
    N = ent_ref.shape[1]
    E = rel_ref.shape[2]
    nb = nb_ref[b]

    rel = rel_ref[0]
    a1 = a1_ref[0]
    a2 = a2_ref[0]
    fr = fr_ref[0]
    fa1 = fa1_ref[0]
    fa2 = fa2_ref[0]
    ent = ent_ref[0]

    hop1 = _mm(rel, w1_ref[...])
    hop2 = _mm(rel, w2_ref[...])

    inf = jnp.float32(jnp.inf)

    def sq(x):
        return jnp.sum(x * x)

    def d2_row(x, fmat, fsq_row):
        return jnp.maximum(fsq_row - 2.0 * _mmt(x, fmat) + sq(x), 0.0)

    def d2_col(x, fmat, fsq_col):
        return jnp.maximum(fsq_col - 2.0 * _mmt(fmat, x) + sq(x), 0.0)

    ones_row = jnp.ones((1, E), jnp.float32)
    fr2_row = _mmt(ones_row, fr * fr)
    fa1sq_row = _mmt(ones_row, fa1 * fa1)
    fa2sq_row = _mmt(ones_row, fa2 * fa2)
    fr2_col = _mmt(fr * fr, ones_row)
    fa1sq_col = _mmt(fa1 * fa1, ones_row)
    fa2sq_col = _mmt(fa2 * fa2, ones_row)

    lane_f = jax.lax.broadcasted_iota(jnp.int32, (1, F), 1)
    sub_f = jax.lax.broadcasted_iota(jnp.int32, (F, 1), 0)
    valid_row = lane_f < nb

    s0 = jnp.maximum(jnp.maximum(d2_row(rel, fr, fr2_row),
                                 d2_row(a1, fa1, fa1sq_row)),
                     d2_row(a2, fa2, fa2sq_row))
    s0 = jnp.where(valid_row, s0, inf)
    score0 = jnp.exp(-0.5 * jnp.min(s0))

    cap1 = jnp.maximum(d2_col(hop1, fr, fr2_col), d2_col(a1, fa1, fa1sq_col))
    cap1 = jnp.where(sub_f < nb, cap1, inf)

    e2_row = _mmt(ones_row, ent * ent)
    dmin = jnp.full((1, N), inf, jnp.float32)
    for t in range(F // _FC):
        fa2c = fa2[t * _FC:(t + 1) * _FC]
        fsqc = fa2sq_col[t * _FC:(t + 1) * _FC]
        capc = cap1[t * _FC:(t + 1) * _FC]
        xf = _mmt(fa2c, ent)
        d2 = jnp.maximum(fsqc + e2_row - 2.0 * xf, 0.0)
        m = jnp.maximum(d2, capc)
        dmin = jnp.minimum(dmin, jnp.min(m, axis=0, keepdims=True))

    vals = jnp.exp(-0.5 * dmin)

    lane_n = jax.lax.broadcasted_iota(jnp.int32, (1, N), 1)
    oh_rows = []
    z_list = []
    v = vals
    for _ in range(_BEAM):
        mv = jnp.max(v)
        idx = jnp.min(jnp.where(v == mv, lane_n, N))
        oh = lane_n == idx
        v = jnp.where(oh, -inf, v)
        z_list.append(mv.reshape(1, 1))
        oh_rows.append(oh.astype(jnp.float32))
    onehot = jnp.concatenate(oh_rows, axis=0)
    z_col = jnp.concatenate(z_list, axis=0)

    zemb = _mm(onehot, ent)

    cap2 = jnp.maximum(d2_row(hop2, fr, fr2_row), d2_row(a2, fa2, fa2sq_row))
    cap2 = jnp.where(valid_row, cap2, inf)
    z2_col = jnp.sum(zemb * zemb, axis=1, keepdims=True)
    xf2 = _mmt(zemb, fa1)
    d2z = jnp.maximum(z2_col + fa1sq_row - 2.0 * xf2, 0.0)
    m2 = jnp.maximum(d2z, cap2)
    h2 = jnp.min(m2, axis=1, keepdims=True)
    sc = jnp.minimum(jnp.exp(-0.5 * h2), z_col)
    res = jnp.max(sc)

    out_ref[...] = jnp.maximum(score0, res).reshape(1, 1, 1)


@jax.jit
def _run(nb_facts, rel, arg1, arg2, fact_rel, fact_arg1, fact_arg2, ent,
         W1, W2):
    B, E = rel.shape
    F = fact_rel.shape[1]
    N = ent.shape[1]
    out = pl.pallas_call(
        _prove_kernel,
        grid=(B,),
        in_specs=[
            pl.BlockSpec(memory_space=pltpu.SMEM),
            pl.BlockSpec((1, 1, E), lambda b: (b, 0, 0)),
            pl.BlockSpec((1, 1, E), lambda b: (b, 0, 0)),
            pl.BlockSpec((1, 1, E), lambda b: (b, 0, 0)),
            pl.BlockSpec((1, F, E), lambda b: (b, 0, 0)),
            pl.BlockSpec((1, F, E), lambda b: (b, 0, 0)),
            pl.BlockSpec((1, F, E), lambda b: (b, 0, 0)),
            pl.BlockSpec((1, N, E), lambda b: (b, 0, 0)),
            pl.BlockSpec((E, E), lambda b: (0, 0)),
            pl.BlockSpec((E, E), lambda b: (0, 0)),
        ],
        out_specs=pl.BlockSpec((1, 1, 1), lambda b: (b, 0, 0)),
        out_shape=jax.ShapeDtypeStruct((B, 1, 1), jnp.float32),
        compiler_params=pltpu.CompilerParams(
            dimension_semantics=("arbitrary",)),
        interpret=_INTERPRET,
    )(nb_facts, rel[:, None, :], arg1[:, None, :], arg2[:, None, :],
      fact_rel, fact_arg1, fact_arg2, ent, W1, W2)
    return out[:, 0, 0]


def kernel(rel, arg1, arg2, fact_rel, fact_arg1, fact_arg2, nb_facts,
           entity_embeddings, nb_entities, W1, W2):
    return _run(nb_facts, rel, arg1, arg2, fact_rel, fact_arg1, fact_arg2,
                entity_embeddings, W1, W2)


# speedup vs baseline: 1.4540x; 1.4540x over previous
import jax
import jax.numpy as jnp
from jax.experimental import pallas as pl
from jax.experimental.pallas import tpu as pltpu

_BEAM = 10
_FC = 256

_INTERPRET = False


def _mmt(a, b):
    return jax.lax.dot_general(a, b, (((1,), (1,)), ((), ())),
                               preferred_element_type=jnp.float32)


def _mm(a, b):
    return jax.lax.dot_general(a, b, (((1,), (0,)), ((), ())),
                               preferred_element_type=jnp.float32)


def _prove_kernel(nb_s_ref, nb_v_ref, rel_ref, a1_ref, a2_ref, fr_ref,
                  fa1_ref, fa2_ref, ent_ref, w1_ref, w2_ref, out_ref):
    B, F, E = fr_ref.shape
    N = ent_ref.shape[1]
    inf = jnp.float32(jnp.inf)

    rel_all = rel_ref[...]
    hop1_all = _mm(rel_all, w1_ref[...])
    hop2_all = _mm(rel_all, w2_ref[...])

    ones_row = jnp.ones((1, E), jnp.float32)
    ones_fcol = jnp.ones((F, 1), jnp.float32)
    sub_f = jax.lax.broadcasted_iota(jnp.int32, (F, 1), 0)

    def sq(x):
        return jnp.sum(x * x)

    def d2_row(x, fmat, fsq_row):
        return jnp.maximum(fsq_row - 2.0 * _mmt(x, fmat) + sq(x), 0.0)

    def d2_col(x, fmat, fsq_col):
        return jnp.maximum(fsq_col - 2.0 * _mmt(fmat, x) + sq(x), 0.0)

    s0_rows = []
    dmin_rows = []
    for b in range(B):
        nb_b = nb_s_ref[b]
        rel_b = rel_all[b:b + 1]
        a1_b = a1_ref[b:b + 1]
        a2_b = a2_ref[b:b + 1]
        hop1_b = hop1_all[b:b + 1]
        hop2_b = hop2_all[b:b + 1]
        fr_b = fr_ref[b]
        fa1_b = fa1_ref[b]
        fa2_b = fa2_ref[b]
        ent_b = ent_ref[b]
        valid_col = sub_f < nb_b

        fr2_row = _mmt(ones_row, fr_b * fr_b)
        fa1sq_row = _mmt(ones_row, fa1_b * fa1_b)
        fa2sq_row = _mmt(ones_row, fa2_b * fa2_b)
        fr2_col = _mmt(fr_b * fr_b, ones_row)
        fa1sq_col = _mmt(fa1_b * fa1_b, ones_row)
        fa2sq_col = _mmt(fa2_b * fa2_b, ones_row)

        s0_rows.append(jnp.maximum(
            jnp.maximum(d2_row(rel_b, fr_b, fr2_row),
                        d2_row(a1_b, fa1_b, fa1sq_row)),
            d2_row(a2_b, fa2_b, fa2sq_row)))

        cap1 = jnp.maximum(d2_col(hop1_b, fr_b, fr2_col),
                           d2_col(a1_b, fa1_b, fa1sq_col))
        cap1 = jnp.where(valid_col, cap1, inf)

        e2_col = _mmt(ent_b * ent_b, ones_row)
        ones_ncol = jnp.ones((N, 1), jnp.float32)
        rhs_aug = jnp.concatenate([ent_b, ones_ncol, e2_col], axis=1)
        lhs_aug = jnp.concatenate([-2.0 * fa2_b, fa2sq_col, ones_fcol],
                                  axis=1)
        dmin = jnp.full((1, N), inf, jnp.float32)
        for t in range(F // _FC):
            pre = _mmt(lhs_aug[t * _FC:(t + 1) * _FC], rhs_aug)
            m = jnp.maximum(pre, cap1[t * _FC:(t + 1) * _FC])
            dmin = jnp.minimum(dmin, jnp.min(m, axis=0, keepdims=True))
        dmin_rows.append(dmin)


    nb_col = nb_v_ref[...]
    lane_f = jax.lax.broadcasted_iota(jnp.int32, (B, F), 1)
    s0_all = jnp.concatenate(s0_rows, axis=0)
    s0_all = jnp.where(lane_f < nb_col, s0_all, inf)
    score0 = jnp.exp(-0.5 * jnp.min(s0_all, axis=1, keepdims=True))

    vals = jnp.exp(-0.5 * jnp.concatenate(dmin_rows, axis=0))

    lane_n = jax.lax.broadcasted_iota(jnp.int32, (B, N), 1)
    ohs = []
    z_cols = []
    v = vals
    for _ in range(_BEAM):
        mv = jnp.max(v, axis=1, keepdims=True)
        idx = jnp.min(jnp.where(v == mv, lane_n, N), axis=1,
                      keepdims=True)
        oh = lane_n == idx
        v = jnp.where(oh, -inf, v)
        z_cols.append(mv)
        ohs.append(oh.astype(jnp.float32))
    z_all = jnp.concatenate(z_cols, axis=1)

    ones_kcol = jnp.ones((_BEAM, 1), jnp.float32)
    sc_rows = []
    for b in range(B):
        nb_b = nb_s_ref[b]
        fr_b = fr_ref[b]
        fa1_b = fa1_ref[b]
        fa2_b = fa2_ref[b]
        hop2_b = hop2_all[b:b + 1]
        a2_b = a2_ref[b:b + 1]
        fr2_col = _mmt(fr_b * fr_b, ones_row)
        fa1sq_col = _mmt(fa1_b * fa1_b, ones_row)
        fa2sq_col = _mmt(fa2_b * fa2_b, ones_row)
        cap2 = jnp.maximum(d2_col(hop2_b, fr_b, fr2_col),
                           d2_col(a2_b, fa2_b, fa2sq_col))
        cap2 = jnp.where(sub_f < nb_b, cap2, inf)
        rhs2_b = jnp.concatenate([fa1_b, ones_fcol, fa1sq_col],
                                 axis=1)
        onehot_b = jnp.concatenate([ohs[j][b:b + 1] for j in range(_BEAM)],
                                   axis=0)
        zemb_b = _mm(onehot_b, ent_ref[b])
        z2_b = _mmt(zemb_b * zemb_b, ones_row)
        lhs2_b = jnp.concatenate([-2.0 * zemb_b, z2_b, ones_kcol],
                                 axis=1)
        pre2 = _mmt(rhs2_b, lhs2_b)
        m2 = jnp.maximum(pre2, cap2)
        h2 = jnp.min(m2, axis=0, keepdims=True)
        sc_rows.append(jnp.minimum(jnp.exp(-0.5 * h2),
                                   z_all[b:b + 1]))
    sc_all = jnp.concatenate(sc_rows, axis=0)
    res = jnp.max(sc_all, axis=1, keepdims=True)

    out_ref[...] = jnp.maximum(score0, res).reshape(B, 1, 1)


@jax.jit
def _run(nb_facts, rel, arg1, arg2, fact_rel, fact_arg1, fact_arg2, ent,
         W1, W2):
    B, E = rel.shape
    F = fact_rel.shape[1]
    N = ent.shape[1]
    full = lambda shape: pl.BlockSpec(shape, lambda i: (0,) * len(shape))
    out = pl.pallas_call(
        _prove_kernel,
        grid=(1,),
        in_specs=[
            pl.BlockSpec(memory_space=pltpu.SMEM),
            full((B, 1)),
            full((B, E)),
            full((B, E)),
            full((B, E)),
            full((B, F, E)),
            full((B, F, E)),
            full((B, F, E)),
            full((B, N, E)),
            full((E, E)),
            full((E, E)),
        ],
        out_specs=full((B, 1, 1)),
        out_shape=jax.ShapeDtypeStruct((B, 1, 1), jnp.float32),
        compiler_params=pltpu.CompilerParams(
            dimension_semantics=("arbitrary",),
            vmem_limit_bytes=64 * 1024 * 1024),
        interpret=_INTERPRET,
    )(nb_facts, nb_facts[:, None], rel, arg1, arg2,
      fact_rel, fact_arg1, fact_arg2, ent, W1, W2)
    return out[:, 0, 0]


def kernel(rel, arg1, arg2, fact_rel, fact_arg1, fact_arg2, nb_facts,
           entity_embeddings, nb_entities, W1, W2):
    return _run(nb_facts, rel, arg1, arg2, fact_rel, fact_arg1, fact_arg2,
                entity_embeddings, W1, W2)


# speedup vs baseline: 2.2669x; 1.5591x over previous
import jax
import jax.numpy as jnp
from jax.experimental import pallas as pl
from jax.experimental.pallas import tpu as pltpu

_BEAM = 10
_FC = 256

_INTERPRET = False


def _mm(a, b):
    return jax.lax.dot_general(a, b, (((1,), (0,)), ((), ())),
                               preferred_element_type=jnp.float32)


def _tm(a, b):
    return jax.lax.dot_general(a, b, (((0,), (0,)), ((), ())),
                               preferred_element_type=jnp.float32)


def _mmt(a, b):
    return jax.lax.dot_general(a, b, (((1,), (1,)), ((), ())),
                               preferred_element_type=jnp.float32)


def _prove_kernel(nb_s_ref, rel_ref, a1_ref, a2_ref, frT_ref,
                  fa1T_ref, fa2T_ref, entT_ref, w1_ref, w2_ref, out_ref,
                  s0_scr, dmin_scr):
    B, E, F = frT_ref.shape
    N = entT_ref.shape[2]
    inf = jnp.float32(jnp.inf)

    ones_row = jnp.ones((1, E), jnp.float32)
    ones_frow = jnp.ones((1, F), jnp.float32)
    ones_nrow = jnp.ones((1, N), jnp.float32)
    lane_f = jax.lax.broadcasted_iota(jnp.int32, (1, F), 1)

    def sq(x):
        return jnp.sum(x * x)

    def d2_row(x, fT, fsq_row):
        return jnp.maximum(fsq_row - 2.0 * _mm(x, fT) + sq(x), 0.0)

    def phase1_body(b, carry):
        nb_b = nb_s_ref[b]
        rel_b = rel_ref[b]
        a1_b = a1_ref[b]
        a2_b = a2_ref[b]
        hop1_b = _mm(rel_b, w1_ref[...])
        frT_b = frT_ref[b]
        fa1T_b = fa1T_ref[b]
        fa2T_b = fa2T_ref[b]
        entT_b = entT_ref[b]
        valid_row = lane_f < nb_b

        fr2_row = _mm(ones_row, frT_b * frT_b)
        fa1sq_row = _mm(ones_row, fa1T_b * fa1T_b)
        fa2sq_row = _mm(ones_row, fa2T_b * fa2T_b)

        s0_row = jnp.maximum(
            jnp.maximum(d2_row(rel_b, frT_b, fr2_row),
                        d2_row(a1_b, fa1T_b, fa1sq_row)),
            d2_row(a2_b, fa2T_b, fa2sq_row))
        s0_row = jnp.where(valid_row, s0_row, inf)

        cap1_row = jnp.maximum(d2_row(hop1_b, frT_b, fr2_row),
                               d2_row(a1_b, fa1T_b, fa1sq_row))
        cap1_row = jnp.where(valid_row, cap1_row, inf)
        cap1_col = _tm(cap1_row, jnp.ones((1, 1), jnp.float32))

        e2_row = _mm(ones_row, entT_b * entT_b)
        rhs_aug = jnp.concatenate([entT_b, ones_nrow, e2_row], axis=0)
        lhs_aug = jnp.concatenate([-2.0 * fa2T_b, fa2sq_row, ones_frow],
                                  axis=0)
        dmin = jnp.full((1, N), inf, jnp.float32)
        for t in range(F // _FC):
            pre = _tm(lhs_aug[:, t * _FC:(t + 1) * _FC], rhs_aug)
            m = jnp.maximum(pre, cap1_col[t * _FC:(t + 1) * _FC])
            dmin = jnp.minimum(dmin, jnp.min(m, axis=0, keepdims=True))

        s0_scr[b] = s0_row
        dmin_scr[b] = dmin
        return carry

    jax.lax.fori_loop(0, B, phase1_body, 0)
    s0_all = jnp.concatenate([s0_scr[b] for b in range(B)], axis=0)
    dmin_all = jnp.concatenate([dmin_scr[b] for b in range(B)], axis=0)
    score0 = jnp.exp(-0.5 * jnp.min(s0_all, axis=1, keepdims=True))

    vals = jnp.exp(-0.5 * dmin_all)

    lane_n = jax.lax.broadcasted_iota(jnp.int32, (B, N), 1)
    ohs = []
    z_cols = []
    v = vals
    for _ in range(_BEAM):
        mv = jnp.max(v, axis=1, keepdims=True)
        idx = jnp.min(jnp.where(v == mv, lane_n, N), axis=1,
                      keepdims=True)
        oh = lane_n == idx
        v = jnp.where(oh, -inf, v)
        z_cols.append(mv)
        ohs.append(oh.astype(jnp.float32))

    ones_krow = jnp.ones((1, _BEAM), jnp.float32)
    m2_rows = []
    z_parts = []
    for b in range(B):
        nb_b = nb_s_ref[b]
        frT_b = frT_ref[b]
        fa1T_b = fa1T_ref[b]
        fa2T_b = fa2T_ref[b]
        entT_b = entT_ref[b]
        hop2_b = _mm(rel_ref[b], w2_ref[...])
        a2_b = a2_ref[b]
        fr2_row = _mm(ones_row, frT_b * frT_b)
        fa1sq_row = _mm(ones_row, fa1T_b * fa1T_b)
        fa2sq_row = _mm(ones_row, fa2T_b * fa2T_b)
        cap2_row = jnp.maximum(d2_row(hop2_b, frT_b, fr2_row),
                               d2_row(a2_b, fa2T_b, fa2sq_row))
        cap2_row = jnp.where(lane_f < nb_b, cap2_row, inf)

        onehot_b = jnp.concatenate([ohs[j][b:b + 1] for j in range(_BEAM)],
                                   axis=0)
        zembT_b = _mmt(entT_b, onehot_b)
        z2_row = _mm(ones_row, zembT_b * zembT_b)
        lhs2 = jnp.concatenate([-2.0 * zembT_b, z2_row, ones_krow],
                               axis=0)
        rhs2 = jnp.concatenate([fa1T_b, ones_frow, fa1sq_row],
                               axis=0)
        pre2 = _tm(lhs2, rhs2)
        m2_rows.append(jnp.maximum(pre2, cap2_row))
        z_parts.extend(z_cols[j][b:b + 1] for j in range(_BEAM))

    m2_all = jnp.concatenate(m2_rows, axis=0)
    h2 = jnp.min(m2_all, axis=1, keepdims=True)
    z80 = jnp.concatenate(z_parts, axis=0)
    sc = jnp.minimum(jnp.exp(-0.5 * h2), z80)
    res_parts = [jnp.max(sc[b * _BEAM:(b + 1) * _BEAM]).reshape(1, 1)
                 for b in range(B)]
    res = jnp.concatenate(res_parts, axis=0)

    out_ref[...] = jnp.maximum(score0, res).reshape(B, 1, 1)


@jax.jit
def _run(nb_facts, rel, arg1, arg2, fact_rel, fact_arg1, fact_arg2, ent,
         W1, W2):
    B, E = rel.shape
    F = fact_rel.shape[1]
    N = ent.shape[1]
    full = lambda shape: pl.BlockSpec(shape, lambda i: (0,) * len(shape))
    out = pl.pallas_call(
        _prove_kernel,
        grid=(1,),
        in_specs=[
            pl.BlockSpec(memory_space=pltpu.SMEM),
            full((B, 1, E)),
            full((B, 1, E)),
            full((B, 1, E)),
            full((B, E, F)),
            full((B, E, F)),
            full((B, E, F)),
            full((B, E, N)),
            full((E, E)),
            full((E, E)),
        ],
        out_specs=full((B, 1, 1)),
        out_shape=jax.ShapeDtypeStruct((B, 1, 1), jnp.float32),
        scratch_shapes=[
            pltpu.VMEM((B, 1, F), jnp.float32),
            pltpu.VMEM((B, 1, N), jnp.float32),
        ],
        compiler_params=pltpu.CompilerParams(
            dimension_semantics=("arbitrary",),
            vmem_limit_bytes=52 * 1024 * 1024),
        interpret=_INTERPRET,
    )(nb_facts, rel[:, None, :], arg1[:, None, :], arg2[:, None, :],
      fact_rel.transpose(0, 2, 1), fact_arg1.transpose(0, 2, 1),
      fact_arg2.transpose(0, 2, 1), ent.transpose(0, 2, 1), W1, W2)
    return out[:, 0, 0]


def kernel(rel, arg1, arg2, fact_rel, fact_arg1, fact_arg2, nb_facts,
           entity_embeddings, nb_entities, W1, W2):
    return _run(nb_facts, rel, arg1, arg2, fact_rel, fact_arg1, fact_arg2,
                entity_embeddings, W1, W2)
